# Initial kernel scaffold; baseline (speedup 1.0000x reference)
import jax
import jax.numpy as jnp
from jax.experimental import pallas as pl


def kernel(x, mask_embed, mask_idx):
    raise NotImplementedError("write your pallas kernel here")


# speedup vs baseline: 1.6723x; 1.6723x over previous
import math

import jax
import jax.numpy as jnp
import numpy as np
from jax.experimental import pallas as pl

D_MODEL = 1024
SEQ_LEN = 4096
BATCH = 4
SEQ_TILE = 512


def _pe_const():
    position = np.arange(SEQ_LEN, dtype=np.float32)[:, None]
    div_term = np.exp(
        np.arange(0, D_MODEL, 2).astype(np.float32) * (-math.log(10000.0) / D_MODEL)
    )
    pe = np.zeros((SEQ_LEN, D_MODEL), dtype=np.float32)
    pe[:, 0::2] = np.sin(position * div_term)
    pe[:, 1::2] = np.cos(position * div_term)
    return pe


def _body(x_ref, pe_ref, f_ref, me_ref, o_ref):
    me0 = me_ref[0:1, :]
    dme = me_ref[1:2, :] - me0
    add = pe_ref[...] + me0 + f_ref[...] * dme
    o_ref[...] = x_ref[...] + add[None]


def kernel(x, mask_embed, mask_idx):
    pe = jnp.asarray(_pe_const())
    f = mask_idx.astype(jnp.float32).reshape(SEQ_LEN, 1)
    grid = (SEQ_LEN // SEQ_TILE,)
    return pl.pallas_call(
        _body,
        grid=grid,
        in_specs=[
            pl.BlockSpec((BATCH, SEQ_TILE, D_MODEL), lambda i: (0, i, 0)),
            pl.BlockSpec((SEQ_TILE, D_MODEL), lambda i: (i, 0)),
            pl.BlockSpec((SEQ_TILE, 1), lambda i: (i, 0)),
            pl.BlockSpec((2, D_MODEL), lambda i: (0, 0)),
        ],
        out_specs=pl.BlockSpec((BATCH, SEQ_TILE, D_MODEL), lambda i: (0, i, 0)),
        out_shape=jax.ShapeDtypeStruct((BATCH, SEQ_LEN, D_MODEL), jnp.float32),
    )(x, pe, f, mask_embed)
